# Initial kernel scaffold; baseline (speedup 1.0000x reference)
import jax
import jax.numpy as jnp
from jax.experimental import pallas as pl


def kernel(x, edge_index, graph_ids, W0, b0, gamma0, beta0, W1, b1, gamma1, beta1, W2, b2, gamma2, beta2, W_ng, b_ng, W_t, b_t):
    raise NotImplementedError("write your pallas kernel here")


# speedup vs baseline: 3.9148x; 3.9148x over previous
import functools

import jax
import jax.numpy as jnp
from jax import lax
from jax.experimental import pallas as pl
from jax.experimental.pallas import tpu as pltpu
from jax.experimental.pallas import tpu_sc as plsc

N = 10000
E = 320000
D = 128
MAX_DEG = 10
G = 64
EPS = 1e-5

NC = 2
NS = 16
CHUNK = 128
CPT = 79
EPAD = NC * NS * CPT * CHUNK
NROWS = 10240
RPT = NROWS // NS

_mesh = plsc.VectorSubcoreMesh(core_axis_name="c", subcore_axis_name="s")


def _msg_body(h_hbm, src_hbm, dst_hbm, zeros_hbm, out_hbm, idx_s, idx_d, rows, sem, accum):
  c = lax.axis_index("c")
  s = lax.axis_index("s")
  pltpu.sync_copy(zeros_hbm, accum.at[pl.ds(s * RPT, RPT)])
  plsc.subcore_barrier()
  wid = c * NS + s

  def body(i, carry):
    row = wid * CPT + i
    pltpu.sync_copy(src_hbm.at[row], idx_s)
    pltpu.sync_copy(dst_hbm.at[row], idx_d)
    pltpu.async_copy(h_hbm.at[idx_s], rows, sem).wait()
    pltpu.sync_copy(rows, accum.at[idx_d], add=True)
    return carry

  lax.fori_loop(0, CPT, body, 0)
  plsc.subcore_barrier()
  pltpu.sync_copy(accum.at[pl.ds(s * RPT, RPT)], out_hbm.at[c, pl.ds(s * RPT, RPT)])


_msg_call = pl.kernel(
    _msg_body,
    out_type=jax.ShapeDtypeStruct((NC, NROWS, D), jnp.float32),
    mesh=_mesh,
    scratch_types=[
        pltpu.VMEM((CHUNK,), jnp.int32),
        pltpu.VMEM((CHUNK,), jnp.int32),
        pltpu.VMEM((CHUNK, D), jnp.float32),
        pltpu.SemaphoreType.DMA,
        pltpu.VMEM_SHARED((NROWS, D), jnp.float32),
    ],
)


def _layer_tc(h_ref, m_ref, cnt_ref, w_ref, b_ref, g_ref, bt_ref, o_ref):
  agg = h_ref[...] + m_ref[0, :N, :] + m_ref[1, :N, :]
  cnt = cnt_ref[0, :N, 0:1] + cnt_ref[1, :N, 0:1]
  deg = jnp.clip(cnt, 1.0, float(MAX_DEG))
  acc = jnp.zeros((N, D), jnp.float32)
  for d in range(MAX_DEG):
    mask = (deg == float(d + 1)).astype(jnp.float32)
    acc = acc + mask * (
        jnp.dot(agg, w_ref[d], preferred_element_type=jnp.float32)
        + b_ref[d : d + 1, :]
    )
  out = jnp.maximum(acc, 0.0)
  mean = jnp.mean(out, axis=0, keepdims=True)
  ctr = out - mean
  var = jnp.mean(ctr * ctr, axis=0, keepdims=True)
  o_ref[...] = g_ref[...] * ctr * lax.rsqrt(var + EPS) + bt_ref[...]


def _layer_call(h, m, cnt, w, b, g, bt):
  return pl.pallas_call(
      _layer_tc,
      out_shape=jax.ShapeDtypeStruct((N, D), jnp.float32),
  )(h, m, cnt, w, b, g, bt)


def _readout_tc(h_ref, gr_ref, gc_ref, wng_ref, bng_ref, wt_ref, bt_ref, o_ref,
                smax_ref):
  t = (
      jnp.dot(h_ref[...], wng_ref[...], preferred_element_type=jnp.float32)
      + bng_ref[...]
  )
  onehot = (
      lax.broadcasted_iota(jnp.int32, (G, N), 0) == gr_ref[...]
  ).astype(jnp.float32)
  ssum = jnp.dot(onehot, t, preferred_element_type=jnp.float32)
  gid_col = gc_ref[...]

  def body(g, carry):
    vals = jnp.where(gid_col == g, t, -jnp.inf)
    smax_ref[pl.ds(g, 1), :] = jnp.max(vals, axis=0, keepdims=True)
    return carry

  lax.fori_loop(0, G, body, 0)
  gfeat = jnp.tanh(jnp.concatenate([ssum, smax_ref[...]], axis=1))
  o_ref[...] = (
      jnp.dot(gfeat, wt_ref[...], preferred_element_type=jnp.float32)
      + bt_ref[...]
  )


def _readout_call(h, gr, gc, wng, bng, wt, bt):
  return pl.pallas_call(
      _readout_tc,
      out_shape=jax.ShapeDtypeStruct((G, 2 * D), jnp.float32),
      scratch_shapes=[pltpu.VMEM((G, D), jnp.float32)],
  )(h, gr, gc, wng, bng, wt, bt)


def kernel(x, edge_index, graph_ids, W0, b0, gamma0, beta0, W1, b1, gamma1,
           beta1, W2, b2, gamma2, beta2, W_ng, b_ng, W_t, b_t):
  src = edge_index[0]
  dst = edge_index[1]
  pad = EPAD - E
  pad_src = (jnp.arange(pad, dtype=jnp.int32) * 13) % N
  pad_dst = N + jnp.arange(pad, dtype=jnp.int32) % (NROWS - N)
  src2 = jnp.concatenate([src, pad_src]).reshape(NC * NS * CPT, CHUNK)
  dst2 = jnp.concatenate([dst, pad_dst]).reshape(NC * NS * CPT, CHUNK)

  zeros_f = jnp.zeros((RPT, D), jnp.float32)
  ones_f = jnp.ones((N, D), jnp.float32)

  cnt = _msg_call(ones_f, src2, dst2, zeros_f)

  h = x
  for (w, b, g, bt) in ((W0, b0, gamma0, beta0), (W1, b1, gamma1, beta1),
                        (W2, b2, gamma2, beta2)):
    m = _msg_call(h, src2, dst2, zeros_f)
    h = _layer_call(h, m, cnt, w, b, g.reshape(1, D), bt.reshape(1, D))

  return _readout_call(
      h,
      graph_ids.reshape(1, N),
      graph_ids.reshape(N, 1),
      W_ng,
      b_ng.reshape(1, D),
      W_t,
      b_t.reshape(1, 2 * D),
  )


# speedup vs baseline: 6.5986x; 1.6855x over previous
import functools

import jax
import jax.numpy as jnp
from jax import lax
from jax.experimental import pallas as pl
from jax.experimental.pallas import tpu as pltpu
from jax.experimental.pallas import tpu_sc as plsc

N = 10000
E = 320000
D = 128
MAX_DEG = 10
G = 64
EPS = 1e-5

NC = 2
NS = 16
CHUNK = 80
CPT = 128
EPAD = NC * NS * CPT * CHUNK
NROWS = 10240
RPT = NROWS // NS
NBUF = 4

_mesh = plsc.VectorSubcoreMesh(core_axis_name="c", subcore_axis_name="s")


def _msg_body(h_hbm, src_hbm, dst_hbm, zeros_hbm, out_hbm, *rest):
  idxs = rest[:NBUF]
  idxd = rest[NBUF:2 * NBUF]
  rows = rest[2 * NBUF:3 * NBUF]
  semi = rest[3 * NBUF:4 * NBUF]
  semg = rest[4 * NBUF:5 * NBUF]
  accum = rest[5 * NBUF]
  c = lax.axis_index("c")
  s = lax.axis_index("s")
  wid = c * NS + s
  for b in range(NBUF):
    pltpu.async_copy(src_hbm.at[wid * CPT + b], idxs[b], semi[b])
    pltpu.async_copy(dst_hbm.at[wid * CPT + b], idxd[b], semi[b])
  pltpu.sync_copy(zeros_hbm, accum.at[pl.ds(s * RPT, RPT)])
  plsc.subcore_barrier()

  def body(j, carry):
    base = j * NBUF
    for b in range(NBUF):
      pltpu.make_async_copy(src_hbm.at[wid * CPT], idxs[b], semi[b]).wait()
      pltpu.make_async_copy(dst_hbm.at[wid * CPT], idxd[b], semi[b]).wait()
      pltpu.async_copy(h_hbm.at[idxs[b]], rows[b], semg[b])
    for b in range(NBUF):
      pltpu.make_async_copy(h_hbm.at[idxs[b]], rows[b], semg[b]).wait()
      pltpu.sync_copy(rows[b], accum.at[idxd[b]], add=True)
      nxt = base + b + NBUF

      @pl.when(nxt < CPT)
      def _():
        pltpu.async_copy(src_hbm.at[wid * CPT + nxt], idxs[b], semi[b])
        pltpu.async_copy(dst_hbm.at[wid * CPT + nxt], idxd[b], semi[b])

    return carry

  lax.fori_loop(0, CPT // NBUF, body, 0)
  plsc.subcore_barrier()
  pltpu.sync_copy(accum.at[pl.ds(s * RPT, RPT)], out_hbm.at[c, pl.ds(s * RPT, RPT)])


_msg_call = pl.kernel(
    _msg_body,
    out_type=jax.ShapeDtypeStruct((NC, NROWS, D), jnp.float32),
    mesh=_mesh,
    scratch_types=(
        [pltpu.VMEM((CHUNK,), jnp.int32)] * (2 * NBUF)
        + [pltpu.VMEM((CHUNK, D), jnp.float32)] * NBUF
        + [pltpu.SemaphoreType.DMA] * (2 * NBUF)
        + [pltpu.VMEM_SHARED((NROWS, D), jnp.float32)]
    ),
)


def _cnt_body(dst_hbm, ones_hbm, zeros_hbm, out_hbm, idxd0, idxd1, sem0, sem1,
              ones_v, counts):
  c = lax.axis_index("c")
  s = lax.axis_index("s")
  wid = c * NS + s
  pltpu.async_copy(dst_hbm.at[wid * CPT], idxd0, sem0)
  pltpu.async_copy(dst_hbm.at[wid * CPT + 1], idxd1, sem1)
  pltpu.sync_copy(zeros_hbm, counts.at[pl.ds(s * RPT, RPT)])
  pltpu.sync_copy(ones_hbm, ones_v)
  plsc.subcore_barrier()

  def body(j, carry):
    i = j * 2
    pltpu.make_async_copy(dst_hbm.at[wid * CPT], idxd0, sem0).wait()
    pltpu.sync_copy(ones_v, counts.at[idxd0], add=True)

    @pl.when(i + 2 < CPT)
    def _():
      pltpu.async_copy(dst_hbm.at[wid * CPT + i + 2], idxd0, sem0)

    pltpu.make_async_copy(dst_hbm.at[wid * CPT], idxd1, sem1).wait()
    pltpu.sync_copy(ones_v, counts.at[idxd1], add=True)

    @pl.when(i + 3 < CPT)
    def _():
      pltpu.async_copy(dst_hbm.at[wid * CPT + i + 3], idxd1, sem1)

    return carry

  lax.fori_loop(0, CPT // 2, body, 0)
  plsc.subcore_barrier()
  pltpu.sync_copy(counts.at[pl.ds(s * RPT, RPT)],
                  out_hbm.at[c, pl.ds(s * RPT, RPT)])


_cnt_call = pl.kernel(
    _cnt_body,
    out_type=jax.ShapeDtypeStruct((NC, NROWS, D), jnp.float32),
    mesh=_mesh,
    scratch_types=[
        pltpu.VMEM((CHUNK,), jnp.int32),
        pltpu.VMEM((CHUNK,), jnp.int32),
        pltpu.SemaphoreType.DMA,
        pltpu.SemaphoreType.DMA,
        pltpu.VMEM((CHUNK, D), jnp.float32),
        pltpu.VMEM_SHARED((NROWS, D), jnp.float32),
    ],
)


def _layer_tc(h_ref, m_ref, cnt_ref, w_ref, b_ref, g_ref, bt_ref, o_ref):
  agg = h_ref[...] + m_ref[0, :N, :] + m_ref[1, :N, :]
  cnt = cnt_ref[0, :N, 0:1] + cnt_ref[1, :N, 0:1]
  deg = jnp.clip(cnt, 1.0, float(MAX_DEG))
  acc = jnp.zeros((N, D), jnp.float32)
  for d in range(MAX_DEG):
    mask = (deg == float(d + 1)).astype(jnp.float32)
    acc = acc + mask * (
        jnp.dot(agg, w_ref[d], preferred_element_type=jnp.float32)
        + b_ref[d : d + 1, :]
    )
  out = jnp.maximum(acc, 0.0)
  mean = jnp.mean(out, axis=0, keepdims=True)
  ctr = out - mean
  var = jnp.mean(ctr * ctr, axis=0, keepdims=True)
  o_ref[...] = g_ref[...] * ctr * lax.rsqrt(var + EPS) + bt_ref[...]


def _layer_call(h, m, cnt, w, b, g, bt):
  return pl.pallas_call(
      _layer_tc,
      out_shape=jax.ShapeDtypeStruct((N, D), jnp.float32),
  )(h, m, cnt, w, b, g, bt)


def _readout_tc(h_ref, gr_ref, gc_ref, wng_ref, bng_ref, wt_ref, bt_ref, o_ref,
                smax_ref):
  t = (
      jnp.dot(h_ref[...], wng_ref[...], preferred_element_type=jnp.float32)
      + bng_ref[...]
  )
  onehot = (
      lax.broadcasted_iota(jnp.int32, (G, N), 0) == gr_ref[...]
  ).astype(jnp.float32)
  ssum = jnp.dot(onehot, t, preferred_element_type=jnp.float32)
  gid_col = gc_ref[...]

  def body(g, carry):
    vals = jnp.where(gid_col == g, t, -jnp.inf)
    smax_ref[pl.ds(g, 1), :] = jnp.max(vals, axis=0, keepdims=True)
    return carry

  lax.fori_loop(0, G, body, 0)
  gfeat = jnp.tanh(jnp.concatenate([ssum, smax_ref[...]], axis=1))
  o_ref[...] = (
      jnp.dot(gfeat, wt_ref[...], preferred_element_type=jnp.float32)
      + bt_ref[...]
  )


def _readout_call(h, gr, gc, wng, bng, wt, bt):
  return pl.pallas_call(
      _readout_tc,
      out_shape=jax.ShapeDtypeStruct((G, 2 * D), jnp.float32),
      scratch_shapes=[pltpu.VMEM((G, D), jnp.float32)],
  )(h, gr, gc, wng, bng, wt, bt)


def kernel(x, edge_index, graph_ids, W0, b0, gamma0, beta0, W1, b1, gamma1,
           beta1, W2, b2, gamma2, beta2, W_ng, b_ng, W_t, b_t):
  src = edge_index[0]
  dst = edge_index[1]
  pad = EPAD - E
  pad_src = (jnp.arange(pad, dtype=jnp.int32) * 13) % N
  pad_dst = N + jnp.arange(pad, dtype=jnp.int32) % (NROWS - N)
  src2 = jnp.concatenate([src, pad_src]).reshape(NC * NS * CPT, CHUNK)
  dst2 = jnp.concatenate([dst, pad_dst]).reshape(NC * NS * CPT, CHUNK)

  zeros_f = jnp.zeros((RPT, D), jnp.float32)
  ones_f = jnp.ones((CHUNK, D), jnp.float32)

  cnt = _cnt_call(dst2, ones_f, zeros_f)

  h = x
  for (w, b, g, bt) in ((W0, b0, gamma0, beta0), (W1, b1, gamma1, beta1),
                        (W2, b2, gamma2, beta2)):
    m = _msg_call(h, src2, dst2, zeros_f)
    h = _layer_call(h, m, cnt, w, b, g.reshape(1, D), bt.reshape(1, D))

  return _readout_call(
      h,
      graph_ids.reshape(1, N),
      graph_ids.reshape(N, 1),
      W_ng,
      b_ng.reshape(1, D),
      W_t,
      b_t.reshape(1, 2 * D),
  )


# speedup vs baseline: 7.4882x; 1.1348x over previous
import functools

import jax
import jax.numpy as jnp
from jax import lax
from jax.experimental import pallas as pl
from jax.experimental.pallas import tpu as pltpu
from jax.experimental.pallas import tpu_sc as plsc

N = 10000
E = 320000
D = 128
MAX_DEG = 10
G = 64
EPS = 1e-5

NC = 2
NS = 16
CHUNK = 80
CPT = 128
EPAD = NC * NS * CPT * CHUNK
NROWS = 10240
RPT = NROWS // NS
NBUF = 4

_mesh = plsc.VectorSubcoreMesh(core_axis_name="c", subcore_axis_name="s")


def _msg_body(h_hbm, src_hbm, dst_hbm, zeros_hbm, out_hbm, *rest):
  idxs = rest[:2 * NBUF]
  idxd = rest[2 * NBUF:4 * NBUF]
  rows = rest[4 * NBUF:5 * NBUF]
  semi = rest[5 * NBUF:7 * NBUF]
  semg = rest[7 * NBUF:8 * NBUF]
  sems = rest[8 * NBUF:9 * NBUF]
  accum = rest[9 * NBUF]
  c = lax.axis_index("c")
  s = lax.axis_index("s")
  wid = c * NS + s

  def load_idx(k, chunk):
    pltpu.async_copy(src_hbm.at[wid * CPT + chunk], idxs[k], semi[k])
    pltpu.async_copy(dst_hbm.at[wid * CPT + chunk], idxd[k], semi[k])

  def wait_idx(k):
    pltpu.make_async_copy(src_hbm.at[wid * CPT], idxs[k], semi[k]).wait()
    pltpu.make_async_copy(dst_hbm.at[wid * CPT], idxd[k], semi[k]).wait()

  def wait_scat(b, k):
    pltpu.make_async_copy(rows[b], accum.at[idxd[k]], sems[b]).wait()

  for k in range(2 * NBUF):
    load_idx(k, k)
  pltpu.sync_copy(zeros_hbm, accum.at[pl.ds(s * RPT, RPT)])
  plsc.subcore_barrier()

  def body(j, carry):
    base = j * 2 * NBUF
    for b in range(NBUF):
      @pl.when(j > 0)
      def _():
        wait_scat(b, b + NBUF)
        load_idx(b + NBUF, base + NBUF + b)
      wait_idx(b)
      pltpu.async_copy(h_hbm.at[idxs[b]], rows[b], semg[b])
    for b in range(NBUF):
      pltpu.make_async_copy(h_hbm.at[idxs[b]], rows[b], semg[b]).wait()
      pltpu.async_copy(rows[b], accum.at[idxd[b]], sems[b], add=True)
    for b in range(NBUF):
      wait_scat(b, b)
      nxt = base + 2 * NBUF + b

      @pl.when(nxt < CPT)
      def _():
        load_idx(b, nxt)
      wait_idx(b + NBUF)
      pltpu.async_copy(h_hbm.at[idxs[b + NBUF]], rows[b], semg[b])
    for b in range(NBUF):
      pltpu.make_async_copy(h_hbm.at[idxs[b + NBUF]], rows[b], semg[b]).wait()
      pltpu.async_copy(rows[b], accum.at[idxd[b + NBUF]], sems[b], add=True)
    return carry

  lax.fori_loop(0, CPT // (2 * NBUF), body, 0)
  for b in range(NBUF):
    wait_scat(b, b + NBUF)
  plsc.subcore_barrier()
  pltpu.sync_copy(accum.at[pl.ds(s * RPT, RPT)], out_hbm.at[c, pl.ds(s * RPT, RPT)])


_msg_call = pl.kernel(
    _msg_body,
    out_type=jax.ShapeDtypeStruct((NC, NROWS, D), jnp.float32),
    mesh=_mesh,
    scratch_types=(
        [pltpu.VMEM((CHUNK,), jnp.int32)] * (4 * NBUF)
        + [pltpu.VMEM((CHUNK, D), jnp.float32)] * NBUF
        + [pltpu.SemaphoreType.DMA] * (4 * NBUF)
        + [pltpu.VMEM_SHARED((NROWS, D), jnp.float32)]
    ),
)


def _cnt_body(dst_hbm, ones_hbm, zeros_hbm, out_hbm, idxd0, idxd1, sem0, sem1,
              ones_v, counts):
  c = lax.axis_index("c")
  s = lax.axis_index("s")
  wid = c * NS + s
  pltpu.async_copy(dst_hbm.at[wid * CPT], idxd0, sem0)
  pltpu.async_copy(dst_hbm.at[wid * CPT + 1], idxd1, sem1)
  pltpu.sync_copy(zeros_hbm, counts.at[pl.ds(s * RPT, RPT)])
  pltpu.sync_copy(ones_hbm, ones_v)
  plsc.subcore_barrier()

  def body(j, carry):
    i = j * 2
    pltpu.make_async_copy(dst_hbm.at[wid * CPT], idxd0, sem0).wait()
    pltpu.sync_copy(ones_v, counts.at[idxd0], add=True)

    @pl.when(i + 2 < CPT)
    def _():
      pltpu.async_copy(dst_hbm.at[wid * CPT + i + 2], idxd0, sem0)

    pltpu.make_async_copy(dst_hbm.at[wid * CPT], idxd1, sem1).wait()
    pltpu.sync_copy(ones_v, counts.at[idxd1], add=True)

    @pl.when(i + 3 < CPT)
    def _():
      pltpu.async_copy(dst_hbm.at[wid * CPT + i + 3], idxd1, sem1)

    return carry

  lax.fori_loop(0, CPT // 2, body, 0)
  plsc.subcore_barrier()
  pltpu.sync_copy(counts.at[pl.ds(s * RPT, RPT)],
                  out_hbm.at[c, pl.ds(s * RPT, RPT)])


_cnt_call = pl.kernel(
    _cnt_body,
    out_type=jax.ShapeDtypeStruct((NC, NROWS, D), jnp.float32),
    mesh=_mesh,
    scratch_types=[
        pltpu.VMEM((CHUNK,), jnp.int32),
        pltpu.VMEM((CHUNK,), jnp.int32),
        pltpu.SemaphoreType.DMA,
        pltpu.SemaphoreType.DMA,
        pltpu.VMEM((CHUNK, D), jnp.float32),
        pltpu.VMEM_SHARED((NROWS, D), jnp.float32),
    ],
)


def _layer_tc(h_ref, m_ref, cnt_ref, w_ref, b_ref, g_ref, bt_ref, o_ref):
  agg = h_ref[...] + m_ref[0, :N, :] + m_ref[1, :N, :]
  cnt = cnt_ref[0, :N, 0:1] + cnt_ref[1, :N, 0:1]
  deg = jnp.clip(cnt, 1.0, float(MAX_DEG))
  acc = jnp.zeros((N, D), jnp.float32)
  for d in range(MAX_DEG):
    mask = (deg == float(d + 1)).astype(jnp.float32)
    acc = acc + mask * (
        jnp.dot(agg, w_ref[d], preferred_element_type=jnp.float32)
        + b_ref[d : d + 1, :]
    )
  out = jnp.maximum(acc, 0.0)
  mean = jnp.mean(out, axis=0, keepdims=True)
  ctr = out - mean
  var = jnp.mean(ctr * ctr, axis=0, keepdims=True)
  o_ref[...] = g_ref[...] * ctr * lax.rsqrt(var + EPS) + bt_ref[...]


def _layer_call(h, m, cnt, w, b, g, bt):
  return pl.pallas_call(
      _layer_tc,
      out_shape=jax.ShapeDtypeStruct((N, D), jnp.float32),
  )(h, m, cnt, w, b, g, bt)


def _readout_tc(h_ref, gr_ref, gc_ref, wng_ref, bng_ref, wt_ref, bt_ref, o_ref,
                smax_ref):
  t = (
      jnp.dot(h_ref[...], wng_ref[...], preferred_element_type=jnp.float32)
      + bng_ref[...]
  )
  onehot = (
      lax.broadcasted_iota(jnp.int32, (G, N), 0) == gr_ref[...]
  ).astype(jnp.float32)
  ssum = jnp.dot(onehot, t, preferred_element_type=jnp.float32)
  gid_col = gc_ref[...]

  def body(g, carry):
    vals = jnp.where(gid_col == g, t, -jnp.inf)
    smax_ref[pl.ds(g, 1), :] = jnp.max(vals, axis=0, keepdims=True)
    return carry

  lax.fori_loop(0, G, body, 0)
  gfeat = jnp.tanh(jnp.concatenate([ssum, smax_ref[...]], axis=1))
  o_ref[...] = (
      jnp.dot(gfeat, wt_ref[...], preferred_element_type=jnp.float32)
      + bt_ref[...]
  )


def _readout_call(h, gr, gc, wng, bng, wt, bt):
  return pl.pallas_call(
      _readout_tc,
      out_shape=jax.ShapeDtypeStruct((G, 2 * D), jnp.float32),
      scratch_shapes=[pltpu.VMEM((G, D), jnp.float32)],
  )(h, gr, gc, wng, bng, wt, bt)


def kernel(x, edge_index, graph_ids, W0, b0, gamma0, beta0, W1, b1, gamma1,
           beta1, W2, b2, gamma2, beta2, W_ng, b_ng, W_t, b_t):
  src = edge_index[0]
  dst = edge_index[1]
  pad = EPAD - E
  pad_src = (jnp.arange(pad, dtype=jnp.int32) * 13) % N
  pad_dst = N + jnp.arange(pad, dtype=jnp.int32) % (NROWS - N)
  src2 = jnp.concatenate([src, pad_src]).reshape(NC * NS * CPT, CHUNK)
  dst2 = jnp.concatenate([dst, pad_dst]).reshape(NC * NS * CPT, CHUNK)

  zeros_f = jnp.zeros((RPT, D), jnp.float32)
  ones_f = jnp.ones((CHUNK, D), jnp.float32)

  cnt = _cnt_call(dst2, ones_f, zeros_f)

  h = x
  for (w, b, g, bt) in ((W0, b0, gamma0, beta0), (W1, b1, gamma1, beta1),
                        (W2, b2, gamma2, beta2)):
    m = _msg_call(h, src2, dst2, zeros_f)
    h = _layer_call(h, m, cnt, w, b, g.reshape(1, D), bt.reshape(1, D))

  return _readout_call(
      h,
      graph_ids.reshape(1, N),
      graph_ids.reshape(N, 1),
      W_ng,
      b_ng.reshape(1, D),
      W_t,
      b_t.reshape(1, 2 * D),
  )
